# baseline (speedup 1.0000x reference)
import functools

import jax
import jax.numpy as jnp
from jax import lax
from jax.experimental import pallas as pl
from jax.experimental.pallas import tpu as pltpu
from jax.experimental.pallas import tpu_sc as plsc

_B, _T, _D = 8, 512, 256
_MEL = 1536
_TBL = _B * _T
_ZROWS = 32
_NC, _NS = 2, 16
_NW = _NC * _NS
_CHUNK = 64
_CPW = _MEL // _CHUNK // 4


def _idx_body(tgt_ref, gidx_ref):
    dur = tgt_ref[...].astype(jnp.float32)
    r = lax.broadcasted_iota(jnp.int32, (_T, _T), 0)
    c = lax.broadcasted_iota(jnp.int32, (_T, _T), 1)
    tri = (r <= c).astype(jnp.float32)
    cum = lax.dot_general(dur, tri, (((1,), (0,)), ((), ()))).astype(jnp.int32)
    mpos = lax.broadcasted_iota(jnp.int32, (_MEL, _T), 0)
    ones = jnp.ones((1, _T), jnp.float32)
    mel = lax.broadcasted_iota(jnp.int32, (1, _MEL), 1)
    zfill = _TBL + (mel % _ZROWS)
    biota = lax.broadcasted_iota(jnp.int32, (_B, _MEL), 0)
    bsel = lax.broadcasted_iota(jnp.int32, (_B, _T), 0)
    acc = jnp.zeros((_B, _MEL), jnp.int32)
    for b in range(_B):
        cum_b = jnp.sum(jnp.where(bsel == b, cum, 0), axis=0, keepdims=True)
        tot_b = jnp.max(cum_b, axis=-1, keepdims=True)
        cmp = (cum_b <= mpos).astype(jnp.float32)
        idx = lax.dot_general(ones, cmp, (((1,), (1,)), ((), ())))
        row = jnp.where(mel < tot_b, idx.astype(jnp.int32) + b * _T, zfill)
        acc = jnp.where(biota == b, row, acc)
    gidx_ref[...] = acc


def _mm(a, b):
    return lax.dot_general(a.astype(jnp.bfloat16), b.astype(jnp.bfloat16),
                           (((1,), (0,)), ((), ())),
                           preferred_element_type=jnp.float32)


def _conv_shift(h, w_ref, b_ref):
    row = lax.broadcasted_iota(jnp.int32, h.shape, 0) % _T
    hp = jnp.where(row == 0, 0.0, pltpu.roll(h, 1, 0))
    hn = jnp.where(row == _T - 1, 0.0, pltpu.roll(h, h.shape[0] - 1, 0))
    return _mm(hp, w_ref[0]) + _mm(h, w_ref[1]) + _mm(hn, w_ref[2]) + b_ref[...]


def _layer_norm(h, g_ref, be_ref):
    m = jnp.mean(h, axis=-1, keepdims=True)
    v = jnp.mean((h - m) * (h - m), axis=-1, keepdims=True)
    return (h - m) * lax.rsqrt(v + 1e-5) * g_ref[...] + be_ref[...]


def _dpo_body(x_ref, w1_ref, b1_ref, g1_ref, be1_ref, w2_ref, b2_ref, g2_ref,
              be2_ref, lw_ref, lb_ref, dpo_ref):
    x = x_ref[...]
    h = jnp.maximum(_layer_norm(_conv_shift(x, w1_ref, b1_ref), g1_ref, be1_ref), 0.0)
    h = jnp.maximum(_layer_norm(_conv_shift(h, w2_ref, b2_ref), g2_ref, be2_ref), 0.0)
    dpo = lax.dot_general(lw_ref[...], h, (((1,), (1,)), ((), ()))) + lb_ref[...]
    dpo_ref[...] = dpo


def _sc_gather_body(xtab_hbm, gidx_hbm, out_hbm, zbuf, *rest):
    idx_b = rest[:_CPW]
    rows_b = rest[_CPW:2 * _CPW]
    g_sems = rest[2 * _CPW:3 * _CPW]
    zsem = rest[3 * _CPW]
    osem = rest[3 * _CPW + 1]
    wid = lax.axis_index("s") * _NC + lax.axis_index("c")
    b = wid // 4
    q = wid % 4
    zcp = pltpu.async_copy(xtab_hbm.at[pl.ds(_TBL, _ZROWS)], zbuf, zsem)
    bases = [b * _MEL + (4 * j + q) * _CHUNK for j in range(_CPW)]
    icps = [
        pltpu.async_copy(gidx_hbm.at[pl.ds(bases[j], _CHUNK)], idx_b[j], g_sems[j])
        for j in range(_CPW)
    ]
    skips = []
    for j in range(_CPW):
        icps[j].wait()
        first = idx_b[j][pl.ds(0, 16)][0]
        skip = first >= _TBL
        skips.append(skip)

        @pl.when(jnp.logical_not(skip))
        def _(j=j):
            pltpu.async_copy(xtab_hbm.at[idx_b[j]], rows_b[j], g_sems[j])
    zcp.wait()
    for j in range(_CPW):
        @pl.when(skips[j])
        def _(j=j):
            pltpu.async_copy(zbuf, out_hbm.at[pl.ds(bases[j], _ZROWS)], osem)
            pltpu.async_copy(zbuf, out_hbm.at[pl.ds(bases[j] + _ZROWS, _ZROWS)], osem)

        @pl.when(jnp.logical_not(skips[j]))
        def _(j=j):
            pltpu.make_async_copy(xtab_hbm.at[pl.ds(0, _CHUNK)], rows_b[j],
                                  g_sems[j]).wait()
            pltpu.async_copy(rows_b[j], out_hbm.at[pl.ds(bases[j], _CHUNK)], osem)
    for j in range(_CPW):
        pltpu.make_async_copy(xtab_hbm.at[pl.ds(0, _CHUNK)], rows_b[j], osem).wait()


@functools.lru_cache(maxsize=None)
def _build_sc_gather():
    return pl.kernel(
        _sc_gather_body,
        mesh=plsc.VectorSubcoreMesh(core_axis_name="c", subcore_axis_name="s"),
        out_type=jax.ShapeDtypeStruct((_B * _MEL, _D), jnp.float32),
        scratch_types=(
            [pltpu.VMEM((_ZROWS, _D), jnp.float32)]
            + [pltpu.VMEM((_CHUNK,), jnp.int32) for _ in range(_CPW)]
            + [pltpu.VMEM((_CHUNK, _D), jnp.float32) for _ in range(_CPW)]
            + [pltpu.SemaphoreType.DMA for _ in range(_CPW)]
            + [pltpu.SemaphoreType.DMA, pltpu.SemaphoreType.DMA]
        ),
    )


def kernel(x, conv1_W, conv1_b, ln1_g, ln1_b, conv2_W, conv2_b, ln2_g, ln2_b,
           lin_W, lin_b, alpha, target, mel_max_length):
    f32 = jnp.float32
    w1t = jnp.transpose(conv1_W, (2, 1, 0))
    w2t = jnp.transpose(conv2_W, (2, 1, 0))
    b1 = conv1_b.reshape(1, -1)
    b2 = conv2_b.reshape(1, -1)
    g1 = ln1_g.reshape(1, -1)
    be1 = ln1_b.reshape(1, -1)
    g2 = ln2_g.reshape(1, -1)
    be2 = ln2_b.reshape(1, -1)
    lw = lin_W.reshape(1, -1)
    lb = lin_b.reshape(1, 1)
    x4 = x.reshape(_B * _T, _D)

    full3 = lambda *_: (0, 0, 0)
    full2 = lambda *_: (0, 0)

    gidx = pl.pallas_call(
        _idx_body,
        grid=(1,),
        in_specs=[pl.BlockSpec((_B, _T), full2)],
        out_specs=pl.BlockSpec((_B, _MEL), full2),
        out_shape=jax.ShapeDtypeStruct((_B, _MEL), jnp.int32),
    )(target)
    xtab = jnp.concatenate([x4, jnp.zeros((_ZROWS, _D), f32)], axis=0)

    out_flat = _build_sc_gather()(xtab, gidx.reshape(_B * _MEL))
    output = out_flat.reshape(_B, _MEL, _D)

    dpo = pl.pallas_call(
        _dpo_body,
        grid=(4,),
        in_specs=[
            pl.BlockSpec((_B * _T // 4, _D), lambda i: (i, 0)),
            pl.BlockSpec((3, _D, _D), full3),
            pl.BlockSpec((1, _D), full2),
            pl.BlockSpec((1, _D), full2),
            pl.BlockSpec((1, _D), full2),
            pl.BlockSpec((3, _D, _D), full3),
            pl.BlockSpec((1, _D), full2),
            pl.BlockSpec((1, _D), full2),
            pl.BlockSpec((1, _D), full2),
            pl.BlockSpec((1, _D), full2),
            pl.BlockSpec((1, 1), full2),
        ],
        out_specs=pl.BlockSpec((1, _B * _T // 4), lambda i: (0, i)),
        out_shape=jax.ShapeDtypeStruct((1, _B * _T), f32),
    )(x4, w1t, b1, g1, be1, w2t, b2, g2, be2, lw, lb)

    return (output, dpo.reshape(_B, _T))


# speedup vs baseline: 1.0387x; 1.0387x over previous
import functools

import jax
import jax.numpy as jnp
from jax import lax
from jax.experimental import pallas as pl
from jax.experimental.pallas import tpu as pltpu
from jax.experimental.pallas import tpu_sc as plsc

_B, _T, _D = 8, 512, 256
_MEL = 1536
_TBL = _B * _T
_ZROWS = 32
_NC, _NS = 2, 16
_NW = _NC * _NS
_CHUNK = 64
_CPW = _MEL // _CHUNK // 4


def _idx_body(x_ref, tgt_ref, gidx_ref, xtab_ref):
    dur = tgt_ref[...].astype(jnp.float32)
    r = lax.broadcasted_iota(jnp.int32, (_T, _T), 0)
    c = lax.broadcasted_iota(jnp.int32, (_T, _T), 1)
    tri = (r <= c).astype(jnp.float32)
    cum = lax.dot_general(dur, tri, (((1,), (0,)), ((), ()))).astype(jnp.int32)
    mpos = lax.broadcasted_iota(jnp.int32, (_MEL, _T), 0)
    ones = jnp.ones((1, _T), jnp.float32)
    mel = lax.broadcasted_iota(jnp.int32, (1, _MEL), 1)
    zfill = _TBL + (mel % _ZROWS)
    biota = lax.broadcasted_iota(jnp.int32, (_B, _MEL), 0)
    bsel = lax.broadcasted_iota(jnp.int32, (_B, _T), 0)
    acc = jnp.zeros((_B, _MEL), jnp.int32)
    for b in range(_B):
        cum_b = jnp.sum(jnp.where(bsel == b, cum, 0), axis=0, keepdims=True)
        tot_b = jnp.max(cum_b, axis=-1, keepdims=True)
        cmp = (cum_b <= mpos).astype(jnp.float32)
        idx = lax.dot_general(ones, cmp, (((1,), (1,)), ((), ())))
        row = jnp.where(mel < tot_b, idx.astype(jnp.int32) + b * _T, zfill)
        acc = jnp.where(biota == b, row, acc)
    gidx_ref[...] = acc
    xtab_ref[0:_TBL, :] = x_ref[...]
    xtab_ref[_TBL:_TBL + _ZROWS, :] = jnp.zeros((_ZROWS, _D), jnp.float32)


def _mm(a, b):
    return lax.dot_general(a.astype(jnp.bfloat16), b.astype(jnp.bfloat16),
                           (((1,), (0,)), ((), ())),
                           preferred_element_type=jnp.float32)


def _conv_shift(h, w_ref, b_ref):
    row = lax.broadcasted_iota(jnp.int32, h.shape, 0) % _T
    hp = jnp.where(row == 0, 0.0, pltpu.roll(h, 1, 0))
    hn = jnp.where(row == _T - 1, 0.0, pltpu.roll(h, h.shape[0] - 1, 0))
    return _mm(hp, w_ref[0]) + _mm(h, w_ref[1]) + _mm(hn, w_ref[2]) + b_ref[...]


def _layer_norm(h, g_ref, be_ref):
    m = jnp.mean(h, axis=-1, keepdims=True)
    v = jnp.mean((h - m) * (h - m), axis=-1, keepdims=True)
    return (h - m) * lax.rsqrt(v + 1e-5) * g_ref[...] + be_ref[...]


def _dpo_body(x_ref, w1_ref, b1_ref, g1_ref, be1_ref, w2_ref, b2_ref, g2_ref,
              be2_ref, lw_ref, lb_ref, dpo_ref):
    x = x_ref[...]
    h = jnp.maximum(_layer_norm(_conv_shift(x, w1_ref, b1_ref), g1_ref, be1_ref), 0.0)
    h = jnp.maximum(_layer_norm(_conv_shift(h, w2_ref, b2_ref), g2_ref, be2_ref), 0.0)
    dpo = lax.dot_general(lw_ref[...], h, (((1,), (1,)), ((), ()))) + lb_ref[...]
    dpo_ref[...] = dpo


def _sc_gather_body(xtab_hbm, gidx_hbm, out_hbm, zbuf, *rest):
    idx_b = rest[:_CPW]
    rows_b = rest[_CPW:2 * _CPW]
    g_sems = rest[2 * _CPW:3 * _CPW]
    zsem = rest[3 * _CPW]
    osem = rest[3 * _CPW + 1]
    wid = lax.axis_index("s") * _NC + lax.axis_index("c")
    b = wid // 4
    q = wid % 4
    zcp = pltpu.async_copy(xtab_hbm.at[pl.ds(_TBL, _ZROWS)], zbuf, zsem)
    bases = [b * _MEL + (4 * j + q) * _CHUNK for j in range(_CPW)]
    icps = [
        pltpu.async_copy(gidx_hbm.at[pl.ds(bases[j], _CHUNK)], idx_b[j], g_sems[j])
        for j in range(_CPW)
    ]
    skips = []
    for j in range(_CPW):
        icps[j].wait()
        first = idx_b[j][pl.ds(0, 16)][0]
        skip = first >= _TBL
        skips.append(skip)

        @pl.when(jnp.logical_not(skip))
        def _(j=j):
            pltpu.async_copy(xtab_hbm.at[idx_b[j]], rows_b[j], g_sems[j])
    zcp.wait()
    for j in range(_CPW):
        @pl.when(skips[j])
        def _(j=j):
            pltpu.async_copy(zbuf, out_hbm.at[pl.ds(bases[j], _ZROWS)], osem)
            pltpu.async_copy(zbuf, out_hbm.at[pl.ds(bases[j] + _ZROWS, _ZROWS)], osem)

        @pl.when(jnp.logical_not(skips[j]))
        def _(j=j):
            pltpu.make_async_copy(xtab_hbm.at[pl.ds(0, _CHUNK)], rows_b[j],
                                  g_sems[j]).wait()
            pltpu.async_copy(rows_b[j], out_hbm.at[pl.ds(bases[j], _CHUNK)], osem)
    for j in range(_CPW):
        pltpu.make_async_copy(xtab_hbm.at[pl.ds(0, _CHUNK)], rows_b[j], osem).wait()


@functools.lru_cache(maxsize=None)
def _build_sc_gather():
    return pl.kernel(
        _sc_gather_body,
        mesh=plsc.VectorSubcoreMesh(core_axis_name="c", subcore_axis_name="s"),
        out_type=jax.ShapeDtypeStruct((_B * _MEL, _D), jnp.float32),
        scratch_types=(
            [pltpu.VMEM((_ZROWS, _D), jnp.float32)]
            + [pltpu.VMEM((_CHUNK,), jnp.int32) for _ in range(_CPW)]
            + [pltpu.VMEM((_CHUNK, _D), jnp.float32) for _ in range(_CPW)]
            + [pltpu.SemaphoreType.DMA for _ in range(_CPW)]
            + [pltpu.SemaphoreType.DMA, pltpu.SemaphoreType.DMA]
        ),
    )


def kernel(x, conv1_W, conv1_b, ln1_g, ln1_b, conv2_W, conv2_b, ln2_g, ln2_b,
           lin_W, lin_b, alpha, target, mel_max_length):
    f32 = jnp.float32
    w1t = jnp.transpose(conv1_W, (2, 1, 0))
    w2t = jnp.transpose(conv2_W, (2, 1, 0))
    b1 = conv1_b.reshape(1, -1)
    b2 = conv2_b.reshape(1, -1)
    g1 = ln1_g.reshape(1, -1)
    be1 = ln1_b.reshape(1, -1)
    g2 = ln2_g.reshape(1, -1)
    be2 = ln2_b.reshape(1, -1)
    lw = lin_W.reshape(1, -1)
    lb = lin_b.reshape(1, 1)
    x4 = x.reshape(_B * _T, _D)

    full3 = lambda *_: (0, 0, 0)
    full2 = lambda *_: (0, 0)

    gidx, xtab = pl.pallas_call(
        _idx_body,
        grid=(1,),
        in_specs=[
            pl.BlockSpec((_B * _T, _D), full2),
            pl.BlockSpec((_B, _T), full2),
        ],
        out_specs=[
            pl.BlockSpec((_B, _MEL), full2),
            pl.BlockSpec((_TBL + _ZROWS, _D), full2),
        ],
        out_shape=[
            jax.ShapeDtypeStruct((_B, _MEL), jnp.int32),
            jax.ShapeDtypeStruct((_TBL + _ZROWS, _D), f32),
        ],
    )(x4, target)

    out_flat = _build_sc_gather()(xtab, gidx.reshape(_B * _MEL))
    output = out_flat.reshape(_B, _MEL, _D)

    dpo = pl.pallas_call(
        _dpo_body,
        grid=(4,),
        in_specs=[
            pl.BlockSpec((_B * _T // 4, _D), lambda i: (i, 0)),
            pl.BlockSpec((3, _D, _D), full3),
            pl.BlockSpec((1, _D), full2),
            pl.BlockSpec((1, _D), full2),
            pl.BlockSpec((1, _D), full2),
            pl.BlockSpec((3, _D, _D), full3),
            pl.BlockSpec((1, _D), full2),
            pl.BlockSpec((1, _D), full2),
            pl.BlockSpec((1, _D), full2),
            pl.BlockSpec((1, _D), full2),
            pl.BlockSpec((1, 1), full2),
        ],
        out_specs=pl.BlockSpec((1, _B * _T // 4), lambda i: (0, i)),
        out_shape=jax.ShapeDtypeStruct((1, _B * _T), f32),
    )(x4, w1t, b1, g1, be1, w2t, b2, g2, be2, lw, lb)

    return (output, dpo.reshape(_B, _T))
